# baseline (speedup 1.0000x reference)
import functools

import jax
import jax.numpy as jnp
from jax import lax
from jax.experimental import pallas as pl
from jax.experimental.pallas import tpu as pltpu
from jax.experimental.pallas import tpu_sc as plsc

B = 4096
UD = 30
ID = 10
GD, GE = 21, 9
DD, DE = 3026, 10
BB = 512


def _sc_gather(user_indices, item_indices, user_table, item_table):
    info = plsc.get_sparse_core_info()
    nc, ns = info.num_cores, info.num_subcores
    nw = nc * ns
    bpw = B // nw

    mesh = plsc.VectorSubcoreMesh(core_axis_name="c", subcore_axis_name="s")

    @functools.partial(
        pl.kernel,
        mesh=mesh,
        compiler_params=pltpu.CompilerParams(needs_layout_passes=False),
        out_type=[
            jax.ShapeDtypeStruct((B, UD), jnp.float32),
            jax.ShapeDtypeStruct((B, ID), jnp.float32),
        ],
        scratch_types=[
            pltpu.VMEM((bpw,), jnp.int32),
            pltpu.VMEM((bpw,), jnp.int32),
            pltpu.VMEM((bpw, UD), jnp.float32),
            pltpu.VMEM((bpw, ID), jnp.float32),
            pltpu.SemaphoreType.DMA,
            pltpu.SemaphoreType.DMA,
            pltpu.SemaphoreType.DMA,
        ],
    )
    def gather_kernel(uidx_hbm, iidx_hbm, ut_hbm, it_hbm, ue_out, ip_out,
                      uidx_v, iidx_v, ue_v, ip_v, sem_u, sem_i, sem_o):
        wid = lax.axis_index("s") * nc + lax.axis_index("c")
        base = wid * bpw
        pltpu.sync_copy(uidx_hbm.at[pl.ds(base, bpw)], uidx_v)
        pltpu.sync_copy(iidx_hbm.at[pl.ds(base, bpw)], iidx_v)

        def body(k, _):
            uvec = uidx_v[pl.ds(k * 16, 16)]
            ivec = iidx_v[pl.ds(k * 16, 16)]
            lane = lax.iota(jnp.int32, 16)
            for j in range(16):
                u = jnp.sum(jnp.where(lane == j, uvec, 0))
                t = jnp.sum(jnp.where(lane == j, ivec, 0))
                i = k * 16 + j
                pltpu.async_copy(ut_hbm.at[pl.ds(u, 1), :],
                                 ue_v.at[pl.ds(i, 1), :], sem_u)
                pltpu.async_copy(it_hbm.at[pl.ds(t, 1), :],
                                 ip_v.at[pl.ds(i, 1), :], sem_i)
            return ()

        lax.fori_loop(0, bpw // 16, body, ())
        pltpu.make_async_copy(ut_hbm.at[pl.ds(0, bpw), :], ue_v, sem_u).wait()
        pltpu.make_async_copy(it_hbm.at[pl.ds(0, bpw), :], ip_v, sem_i).wait()
        cu = pltpu.async_copy(ue_v, ue_out.at[pl.ds(base, bpw)], sem_o)
        ci = pltpu.async_copy(ip_v, ip_out.at[pl.ds(base, bpw)], sem_o)
        cu.wait()
        ci.wait()

    return gather_kernel(user_indices, item_indices, user_table, item_table)


def _tc_body(ue_ref, ip_ref, b_ref, g_ref, d_ref, Wg_ref, bg_ref, Wd_ref,
             bd_ref, Wa_ref, ba_ref, out_ref):
    f32 = jnp.float32
    c_ip = lax.broadcasted_iota(jnp.int32, (ID, UD), 1)
    r_ip = lax.broadcasted_iota(jnp.int32, (ID, UD), 0)
    E_ip = (c_ip == r_ip).astype(f32)
    c_g = lax.broadcasted_iota(jnp.int32, (GE, UD), 1)
    r_g = lax.broadcasted_iota(jnp.int32, (GE, UD), 0)
    E_g = (c_g == r_g + ID + 1).astype(f32)
    c_d = lax.broadcasted_iota(jnp.int32, (DE, UD), 1)
    r_d = lax.broadcasted_iota(jnp.int32, (DE, UD), 0)
    E_d = (c_d == r_d + ID + 1 + GE).astype(f32)
    onehot_b = (lax.broadcasted_iota(jnp.int32, (1, UD), 1) == ID).astype(f32)

    Wg30 = jnp.dot(Wg_ref[...], E_g, preferred_element_type=f32)
    Wd30 = jnp.dot(Wd_ref[...], E_d, preferred_element_type=f32)
    bias30 = (jnp.dot(bg_ref[...], E_g, preferred_element_type=f32)
              + jnp.dot(bd_ref[...], E_d, preferred_element_type=f32))

    item = (jnp.dot(ip_ref[...], E_ip, preferred_element_type=f32)
            + jnp.dot(g_ref[...], Wg30, preferred_element_type=f32)
            + jnp.dot(d_ref[...], Wd30, preferred_element_type=f32)
            + b_ref[...] * onehot_b
            + bias30)

    prod = ue_ref[...] * item
    logits = jnp.dot(prod, Wa_ref[...], preferred_element_type=f32) + ba_ref[...]
    out_ref[...] = jax.nn.sigmoid(logits)


def _tc_dense(ue, ip, b2, g, d, Wg, bg2, Wd, bd2, Wa, ba2):
    return pl.pallas_call(
        _tc_body,
        grid=(B // BB,),
        in_specs=[
            pl.BlockSpec((BB, UD), lambda i: (i, 0)),
            pl.BlockSpec((BB, ID), lambda i: (i, 0)),
            pl.BlockSpec((BB, 1), lambda i: (i, 0)),
            pl.BlockSpec((BB, GD), lambda i: (i, 0)),
            pl.BlockSpec((BB, DD), lambda i: (i, 0)),
            pl.BlockSpec((GD, GE), lambda i: (0, 0)),
            pl.BlockSpec((1, GE), lambda i: (0, 0)),
            pl.BlockSpec((DD, DE), lambda i: (0, 0)),
            pl.BlockSpec((1, DE), lambda i: (0, 0)),
            pl.BlockSpec((UD, 1), lambda i: (0, 0)),
            pl.BlockSpec((1, 1), lambda i: (0, 0)),
        ],
        out_specs=pl.BlockSpec((BB, 1), lambda i: (i, 0)),
        out_shape=jax.ShapeDtypeStruct((B, 1), jnp.float32),
    )(ue, ip, b2, g, d, Wg, bg2, Wd, bd2, Wa, ba2)


def kernel(user_indices, item_indices, b, g, d, user_table, item_table,
           Wg, bg, Wd, bd, Wa, ba):
    ue, ip = _sc_gather(user_indices.astype(jnp.int32),
                        item_indices.astype(jnp.int32),
                        user_table, item_table)
    out = _tc_dense(ue, ip,
                    b.reshape(B, 1).astype(jnp.float32),
                    g.astype(jnp.float32), d.astype(jnp.float32),
                    Wg, bg.reshape(1, GE), Wd, bd.reshape(1, DE),
                    Wa, ba.reshape(1, 1))
    return out.reshape(B)


# speedup vs baseline: 5.2148x; 5.2148x over previous
import functools

import jax
import jax.numpy as jnp
from jax import lax
from jax.experimental import pallas as pl
from jax.experimental.pallas import tpu as pltpu
from jax.experimental.pallas import tpu_sc as plsc

B = 4096
UD = 30
ID = 10
GD, GE = 21, 9
DD, DE = 3026, 10
BB = 512
LANES = 128


def _sc_gather_t(user_indices, item_indices, ut_t, it_t):
    info = plsc.get_sparse_core_info()
    nc, ns = info.num_cores, info.num_subcores
    nw = nc * ns
    bpw = B // nw
    nchunks = bpw // 16

    mesh = plsc.VectorSubcoreMesh(core_axis_name="c", subcore_axis_name="s")

    @functools.partial(
        pl.kernel,
        mesh=mesh,
        compiler_params=pltpu.CompilerParams(needs_layout_passes=False),
        out_type=[
            jax.ShapeDtypeStruct((UD, B), jnp.float32),
            jax.ShapeDtypeStruct((ID, B), jnp.float32),
        ],
        scratch_types=[
            pltpu.VMEM((bpw,), jnp.int32),
            pltpu.VMEM((bpw,), jnp.int32),
            pltpu.VMEM((16 * UD, LANES), jnp.float32),
            pltpu.VMEM((16 * ID, LANES), jnp.float32),
            pltpu.VMEM((UD, bpw), jnp.float32),
            pltpu.VMEM((ID, bpw), jnp.float32),
            pltpu.SemaphoreType.DMA,
            pltpu.SemaphoreType.DMA,
            pltpu.SemaphoreType.DMA,
        ],
    )
    def gather_kernel(uidx_hbm, iidx_hbm, ut_hbm, it_hbm, ue_out, ip_out,
                      uidx_v, iidx_v, stg_u, stg_i, uet_v, ipt_v,
                      sem_u, sem_i, sem_o):
        wid = lax.axis_index("s") * nc + lax.axis_index("c")
        base = pl.multiple_of(wid * bpw, LANES)
        pltpu.sync_copy(uidx_hbm.at[pl.ds(base, bpw)], uidx_v)
        pltpu.sync_copy(iidx_hbm.at[pl.ds(base, bpw)], iidx_v)
        lane16 = lax.iota(jnp.int32, 16)

        def chunk(c, _):
            uvec = uidx_v[pl.ds(c * 16, 16)]
            ivec = iidx_v[pl.ds(c * 16, 16)]
            copies = []
            for j in range(16):
                u = jnp.sum(jnp.where(lane16 == j, uvec, 0))
                t = jnp.sum(jnp.where(lane16 == j, ivec, 0))
                u_al = pl.multiple_of((u // LANES) * LANES, LANES)
                t_al = pl.multiple_of((t // LANES) * LANES, LANES)
                copies.append(
                    pltpu.async_copy(ut_hbm.at[:, pl.ds(u_al, LANES)],
                                     stg_u.at[pl.ds(j * UD, UD), :], sem_u))
                copies.append(
                    pltpu.async_copy(it_hbm.at[:, pl.ds(t_al, LANES)],
                                     stg_i.at[pl.ds(j * ID, ID), :], sem_i))
            for cp in copies:
                cp.wait()
            ulane = lax.rem(uvec, LANES)
            ilane = lax.rem(ivec, LANES)
            cols = c * 16 + lane16

            def ext_u(r, _):
                rv = jnp.full((16,), r, jnp.int32)
                vals = plsc.load_gather(stg_u, [lane16 * UD + rv, ulane])
                plsc.store_scatter(uet_v, [rv, cols], vals)
                return ()

            def ext_i(r, _):
                rv = jnp.full((16,), r, jnp.int32)
                vals = plsc.load_gather(stg_i, [lane16 * ID + rv, ilane])
                plsc.store_scatter(ipt_v, [rv, cols], vals)
                return ()

            lax.fori_loop(0, UD, ext_u, ())
            lax.fori_loop(0, ID, ext_i, ())
            return ()

        lax.fori_loop(0, nchunks, chunk, ())
        cu = pltpu.async_copy(uet_v, ue_out.at[:, pl.ds(base, bpw)], sem_o)
        ci = pltpu.async_copy(ipt_v, ip_out.at[:, pl.ds(base, bpw)], sem_o)
        cu.wait()
        ci.wait()

    return gather_kernel(user_indices, item_indices, ut_t, it_t)


def _tc_body(uet_ref, ipt_ref, bt_ref, gt_ref, dt_ref, WgT_ref, bg_ref,
             WdT_ref, bd_ref, WaT_ref, ba_ref, out_ref):
    f32 = jnp.float32
    r_ip = lax.broadcasted_iota(jnp.int32, (UD, ID), 0)
    c_ip = lax.broadcasted_iota(jnp.int32, (UD, ID), 1)
    E_ipT = (r_ip == c_ip).astype(f32)
    r_g = lax.broadcasted_iota(jnp.int32, (UD, GE), 0)
    c_g = lax.broadcasted_iota(jnp.int32, (UD, GE), 1)
    E_gT = (r_g == c_g + ID + 1).astype(f32)
    r_d = lax.broadcasted_iota(jnp.int32, (UD, DE), 0)
    c_d = lax.broadcasted_iota(jnp.int32, (UD, DE), 1)
    E_dT = (r_d == c_d + ID + 1 + GE).astype(f32)
    onehot_col = (lax.broadcasted_iota(jnp.int32, (UD, 1), 0) == ID).astype(f32)

    Wg30T = jnp.dot(E_gT, WgT_ref[...], preferred_element_type=f32)
    Wd30T = jnp.dot(E_dT, WdT_ref[...], preferred_element_type=f32)
    bias_col = (jnp.dot(E_gT, bg_ref[...], preferred_element_type=f32)
                + jnp.dot(E_dT, bd_ref[...], preferred_element_type=f32))

    item_t = (jnp.dot(E_ipT, ipt_ref[...], preferred_element_type=f32)
              + jnp.dot(Wg30T, gt_ref[...], preferred_element_type=f32)
              + jnp.dot(Wd30T, dt_ref[...], preferred_element_type=f32)
              + onehot_col * bt_ref[...]
              + bias_col)

    prod_t = uet_ref[...] * item_t
    logits = jnp.dot(WaT_ref[...], prod_t, preferred_element_type=f32) + ba_ref[...]
    out_ref[...] = jax.nn.sigmoid(logits)


def _tc_dense_t(uet, ipt, bt, gt, dt, WgT, bg2, WdT, bd2, WaT, ba2):
    return pl.pallas_call(
        _tc_body,
        grid=(B // BB,),
        in_specs=[
            pl.BlockSpec((UD, BB), lambda i: (0, i)),
            pl.BlockSpec((ID, BB), lambda i: (0, i)),
            pl.BlockSpec((1, BB), lambda i: (0, i)),
            pl.BlockSpec((GD, BB), lambda i: (0, i)),
            pl.BlockSpec((DD, BB), lambda i: (0, i)),
            pl.BlockSpec((GE, GD), lambda i: (0, 0)),
            pl.BlockSpec((GE, 1), lambda i: (0, 0)),
            pl.BlockSpec((DE, DD), lambda i: (0, 0)),
            pl.BlockSpec((DE, 1), lambda i: (0, 0)),
            pl.BlockSpec((1, UD), lambda i: (0, 0)),
            pl.BlockSpec((1, 1), lambda i: (0, 0)),
        ],
        out_specs=pl.BlockSpec((1, BB), lambda i: (0, i)),
        out_shape=jax.ShapeDtypeStruct((1, B), jnp.float32),
    )(uet, ipt, bt, gt, dt, WgT, bg2, WdT, bd2, WaT, ba2)


def kernel(user_indices, item_indices, b, g, d, user_table, item_table,
           Wg, bg, Wd, bd, Wa, ba):
    uet, ipt = _sc_gather_t(user_indices.astype(jnp.int32),
                            item_indices.astype(jnp.int32),
                            user_table.T, item_table.T)
    out = _tc_dense_t(uet, ipt,
                      b.reshape(1, B),
                      g.T, d.T,
                      Wg.T, bg.reshape(GE, 1), Wd.T, bd.reshape(DE, 1),
                      Wa.T, ba.reshape(1, 1))
    return out.reshape(B)


# speedup vs baseline: 5.6328x; 1.0802x over previous
import functools

import jax
import jax.numpy as jnp
from jax import lax
from jax.experimental import pallas as pl
from jax.experimental.pallas import tpu as pltpu
from jax.experimental.pallas import tpu_sc as plsc

B = 4096
UD = 30
ID = 10
GD, GE = 21, 9
DD, DE = 3026, 10
BB = 512
LANES = 128


def _sc_gather_t(user_indices, item_indices, ut_t, it_t):
    info = plsc.get_sparse_core_info()
    nc, ns = info.num_cores, info.num_subcores
    nw = nc * ns
    bpw = B // nw
    nchunks = bpw // 16

    mesh = plsc.VectorSubcoreMesh(core_axis_name="c", subcore_axis_name="s")

    @functools.partial(
        pl.kernel,
        mesh=mesh,
        compiler_params=pltpu.CompilerParams(needs_layout_passes=False),
        out_type=[
            jax.ShapeDtypeStruct((UD, B), jnp.float32),
            jax.ShapeDtypeStruct((ID, B), jnp.float32),
        ],
        scratch_types=[
            pltpu.VMEM((bpw,), jnp.int32),
            pltpu.VMEM((bpw,), jnp.int32),
            pltpu.VMEM((16 * UD, LANES), jnp.float32),
            pltpu.VMEM((16 * ID, LANES), jnp.float32),
            pltpu.VMEM((UD, bpw), jnp.float32),
            pltpu.VMEM((ID, bpw), jnp.float32),
            pltpu.SemaphoreType.DMA,
            pltpu.SemaphoreType.DMA,
            pltpu.SemaphoreType.DMA,
        ],
    )
    def gather_kernel(uidx_hbm, iidx_hbm, ut_hbm, it_hbm, ue_out, ip_out,
                      uidx_v, iidx_v, stg_u, stg_i, uet_v, ipt_v,
                      sem_u, sem_i, sem_o):
        wid = lax.axis_index("s") * nc + lax.axis_index("c")
        base = pl.multiple_of(wid * bpw, LANES)
        pltpu.sync_copy(uidx_hbm.at[pl.ds(base, bpw)], uidx_v)
        pltpu.sync_copy(iidx_hbm.at[pl.ds(base, bpw)], iidx_v)
        lane16 = lax.iota(jnp.int32, 16)

        def chunk(c, _):
            uvec = uidx_v[pl.ds(c * 16, 16)]
            ivec = iidx_v[pl.ds(c * 16, 16)]
            copies = []
            for j in range(16):
                u = jnp.sum(jnp.where(lane16 == j, uvec, 0))
                t = jnp.sum(jnp.where(lane16 == j, ivec, 0))
                u_al = pl.multiple_of((u // LANES) * LANES, LANES)
                t_al = pl.multiple_of((t // LANES) * LANES, LANES)
                copies.append(
                    pltpu.async_copy(ut_hbm.at[:, pl.ds(u_al, LANES)],
                                     stg_u.at[pl.ds(j * UD, UD), :], sem_u))
                copies.append(
                    pltpu.async_copy(it_hbm.at[:, pl.ds(t_al, LANES)],
                                     stg_i.at[pl.ds(j * ID, ID), :], sem_i))
            for cp in copies:
                cp.wait()
            ulane = lax.rem(uvec, LANES)
            ilane = lax.rem(ivec, LANES)
            cols = c * 16 + lane16

            def ext_u(r, _):
                rv = jnp.full((16,), r, jnp.int32)
                vals = plsc.load_gather(stg_u, [lane16 * UD + rv, ulane])
                plsc.store_scatter(uet_v, [rv, cols], vals)
                return ()

            def ext_i(r, _):
                rv = jnp.full((16,), r, jnp.int32)
                vals = plsc.load_gather(stg_i, [lane16 * ID + rv, ilane])
                plsc.store_scatter(ipt_v, [rv, cols], vals)
                return ()

            lax.fori_loop(0, UD, ext_u, ())
            lax.fori_loop(0, ID, ext_i, ())
            return ()

        lax.fori_loop(0, nchunks, chunk, ())
        cu = pltpu.async_copy(uet_v, ue_out.at[:, pl.ds(base, bpw)], sem_o)
        ci = pltpu.async_copy(ipt_v, ip_out.at[:, pl.ds(base, bpw)], sem_o)
        cu.wait()
        ci.wait()

    return gather_kernel(user_indices, item_indices, ut_t, it_t)


def _tc_dense_body(bt_ref, gt_ref, dt_ref, WgT_ref, bg_ref,
                   WdT_ref, bd_ref, out_ref):
    f32 = jnp.float32
    r_g = lax.broadcasted_iota(jnp.int32, (UD, GE), 0)
    c_g = lax.broadcasted_iota(jnp.int32, (UD, GE), 1)
    E_gT = (r_g == c_g + ID + 1).astype(f32)
    r_d = lax.broadcasted_iota(jnp.int32, (UD, DE), 0)
    c_d = lax.broadcasted_iota(jnp.int32, (UD, DE), 1)
    E_dT = (r_d == c_d + ID + 1 + GE).astype(f32)
    onehot_col = (lax.broadcasted_iota(jnp.int32, (UD, 1), 0) == ID).astype(f32)

    Wg30T = jnp.dot(E_gT, WgT_ref[...], preferred_element_type=f32)
    Wd30T = jnp.dot(E_dT, WdT_ref[...], preferred_element_type=f32)
    bias_col = (jnp.dot(E_gT, bg_ref[...], preferred_element_type=f32)
                + jnp.dot(E_dT, bd_ref[...], preferred_element_type=f32))

    out_ref[...] = (jnp.dot(Wg30T, gt_ref[...], preferred_element_type=f32)
                    + jnp.dot(Wd30T, dt_ref[...], preferred_element_type=f32)
                    + onehot_col * bt_ref[...]
                    + bias_col)


def _tc_dense_t(bt, gt, dt, WgT, bg2, WdT, bd2):
    return pl.pallas_call(
        _tc_dense_body,
        grid=(B // BB,),
        in_specs=[
            pl.BlockSpec((1, BB), lambda i: (0, i)),
            pl.BlockSpec((GD, BB), lambda i: (0, i)),
            pl.BlockSpec((DD, BB), lambda i: (0, i)),
            pl.BlockSpec((GE, GD), lambda i: (0, 0)),
            pl.BlockSpec((GE, 1), lambda i: (0, 0)),
            pl.BlockSpec((DE, DD), lambda i: (0, 0)),
            pl.BlockSpec((DE, 1), lambda i: (0, 0)),
        ],
        out_specs=pl.BlockSpec((UD, BB), lambda i: (0, i)),
        out_shape=jax.ShapeDtypeStruct((UD, B), jnp.float32),
    )(bt, gt, dt, WgT, bg2, WdT, bd2)


def _tc_combine_body(uet_ref, ipt_ref, ipart_ref, WaT_ref, ba_ref, out_ref):
    f32 = jnp.float32
    r_ip = lax.broadcasted_iota(jnp.int32, (UD, ID), 0)
    c_ip = lax.broadcasted_iota(jnp.int32, (UD, ID), 1)
    E_ipT = (r_ip == c_ip).astype(f32)
    item_t = ipart_ref[...] + jnp.dot(E_ipT, ipt_ref[...],
                                      preferred_element_type=f32)
    prod_t = uet_ref[...] * item_t
    logits = (jnp.dot(WaT_ref[...], prod_t, preferred_element_type=f32)
              + ba_ref[...])
    out_ref[...] = jax.nn.sigmoid(logits)


def _tc_combine(uet, ipt, ipart, WaT, ba2):
    return pl.pallas_call(
        _tc_combine_body,
        grid=(B // 2048,),
        in_specs=[
            pl.BlockSpec((UD, 2048), lambda i: (0, i)),
            pl.BlockSpec((ID, 2048), lambda i: (0, i)),
            pl.BlockSpec((UD, 2048), lambda i: (0, i)),
            pl.BlockSpec((1, UD), lambda i: (0, 0)),
            pl.BlockSpec((1, 1), lambda i: (0, 0)),
        ],
        out_specs=pl.BlockSpec((1, 2048), lambda i: (0, i)),
        out_shape=jax.ShapeDtypeStruct((1, B), jnp.float32),
    )(uet, ipt, ipart, WaT, ba2)


def kernel(user_indices, item_indices, b, g, d, user_table, item_table,
           Wg, bg, Wd, bd, Wa, ba):
    uet, ipt = _sc_gather_t(user_indices.astype(jnp.int32),
                            item_indices.astype(jnp.int32),
                            user_table.T, item_table.T)
    ipart = _tc_dense_t(b.reshape(1, B), g.T, d.T,
                        Wg.T, bg.reshape(GE, 1), Wd.T, bd.reshape(DE, 1))
    out = _tc_combine(uet, ipt, ipart, Wa.T, ba.reshape(1, 1))
    return out.reshape(B)
